# baseline (speedup 1.0000x reference)
import functools

import jax
import jax.numpy as jnp
import numpy as np
from jax import lax
from jax.experimental import pallas as pl
from jax.experimental.pallas import tpu as pltpu
from jax.experimental.pallas import tpu_sc as plsc

B = 4096
W = 128
_TINY = np.float32(np.finfo(np.float32).tiny)


def _rotl(x, d):
    return lax.shift_left(x, np.int32(d)) | lax.shift_right_logical(
        x, np.int32(32 - d)
    )


def _threefry_bits(p, seed):
    ks0 = np.int32(0)
    ks1 = np.int32(seed)
    ks2 = np.int32(0x1BD11BDA ^ seed)
    x0 = jnp.zeros_like(p)
    x1 = p + ks1

    def four(x0, x1, rots):
        for r in rots:
            x0 = x0 + x1
            x1 = _rotl(x1, r)
            x1 = x1 ^ x0
        return x0, x1

    r0 = (13, 15, 26, 6)
    r1 = (17, 29, 16, 24)
    x0, x1 = four(x0, x1, r0)
    x0, x1 = x0 + ks1, x1 + (ks2 + np.int32(1))
    x0, x1 = four(x0, x1, r1)
    x0, x1 = x0 + ks2, x1 + (ks0 + np.int32(2))
    x0, x1 = four(x0, x1, r0)
    x0, x1 = x0 + ks0, x1 + (ks1 + np.int32(3))
    x0, x1 = four(x0, x1, r1)
    x0, x1 = x0 + ks1, x1 + (ks2 + np.int32(4))
    x0, x1 = four(x0, x1, r0)
    x0, x1 = x0 + ks2, x1 + (ks0 + np.int32(5))
    return x0 ^ x1


def _gumbel(bits):
    fb = lax.shift_right_logical(bits, np.int32(9)) | np.int32(0x3F800000)
    floats = lax.bitcast_convert_type(fb, jnp.float32) - np.float32(1.0)
    u = jnp.maximum(_TINY, floats * (np.float32(1.0) - _TINY) + _TINY)
    return -jnp.log(-jnp.log(u))


def _softplus(x):
    return jnp.maximum(x, 0.0) + jnp.log1p(jnp.exp(-jnp.abs(x)))


def _neg_side(d, mask, j, i0, br):
    r = lax.broadcasted_iota(jnp.int32, (br, B), 0) + i0 * np.int32(br)
    p = lax.shift_left(r, np.int32(12)) + j
    gn = _gumbel(_threefry_bits(p, 456))
    neg_inf = np.float32(-np.inf)
    score = jnp.where(mask, neg_inf, gn + (-d))
    m = jnp.max(score, axis=1, keepdims=True)
    idx = jnp.min(jnp.where(score == m, j, np.int32(B)), axis=1, keepdims=True)
    return jnp.max(jnp.where(j == idx, d, neg_inf), axis=1, keepdims=True)


def _dense_body(dist_ref, pid_row_ref, pid_all_ref, out_ref, *, br):
    i0 = pl.program_id(0)
    d = dist_ref[...]
    mask = pid_row_ref[...] == pid_all_ref[...]
    j = lax.broadcasted_iota(jnp.int32, (br, B), 1)

    r = lax.broadcasted_iota(jnp.int32, (br, B), 0) + i0 * np.int32(br)
    p = lax.shift_left(r, np.int32(12)) + j
    neg_inf = np.float32(-np.inf)
    gp = _gumbel(_threefry_bits(p, 123))
    score_p = jnp.where(mask, gp + d, neg_inf)
    m = jnp.max(score_p, axis=1, keepdims=True)
    idx = jnp.min(
        jnp.where(score_p == m, j, np.int32(B)), axis=1, keepdims=True
    )
    vp = jnp.max(jnp.where(j == idx, d, neg_inf), axis=1, keepdims=True)

    vn = _neg_side(d, mask, j, i0, br)
    out_ref[...] = _softplus(vp - vn)


def _dense_path(dist, pid_col, pid_all):
    br = 256
    body = functools.partial(_dense_body, br=br)
    return pl.pallas_call(
        body,
        grid=(B // br,),
        in_specs=[
            pl.BlockSpec((br, B), lambda i: (i, 0)),
            pl.BlockSpec((br, 1), lambda i: (i, 0)),
            pl.BlockSpec((1, B), lambda i: (0, 0)),
        ],
        out_specs=pl.BlockSpec((br, 1), lambda i: (i, 0)),
        out_shape=jax.ShapeDtypeStruct((B, 1), jnp.float32),
    )(dist, pid_col, pid_all)


_NW = 32
_ROWS_PER_W = B // _NW
_CR = 16


def _sc_gather_body(dist_hbm, order_hbm, gidx_hbm, col_hbm, dval_hbm,
                    rowbuf, order_buf, gbuf, cbuf, dbuf):
    wid = lax.axis_index("s") * 2 + lax.axis_index("c")
    lane = lax.iota(jnp.int32, 16)
    pltpu.sync_copy(order_hbm, order_buf)
    pltpu.sync_copy(gidx_hbm.at[wid], gbuf)

    def chunk(c, _):
        r0 = wid * np.int32(_ROWS_PER_W) + c * np.int32(_CR)
        pltpu.sync_copy(dist_hbm.at[pl.ds(r0, _CR)], rowbuf)
        c16 = c * np.int32(_CR)

        def cand(k, _):
            g = gbuf[k, pl.ds(c16, _CR)]
            col = plsc.load_gather(order_buf, [g])
            dv = plsc.load_gather(rowbuf, [lane, col])
            cbuf[k, pl.ds(c16, _CR)] = col
            dbuf[k, pl.ds(c16, _CR)] = dv
            return 0

        lax.fori_loop(0, W, cand, 0, unroll=8)
        return 0

    lax.fori_loop(0, _ROWS_PER_W // _CR, chunk, 0)
    pltpu.sync_copy(cbuf, col_hbm.at[wid])
    pltpu.sync_copy(dbuf, dval_hbm.at[wid])


def _sc_gather(dist, order, gidx3):
    mesh = plsc.VectorSubcoreMesh(core_axis_name="c", subcore_axis_name="s")
    f = functools.partial(
        pl.kernel,
        mesh=mesh,
        compiler_params=pltpu.CompilerParams(needs_layout_passes=False),
        out_type=[
            jax.ShapeDtypeStruct((_NW, W, _ROWS_PER_W), jnp.int32),
            jax.ShapeDtypeStruct((_NW, W, _ROWS_PER_W), jnp.float32),
        ],
        scratch_types=[
            pltpu.VMEM((_CR, B), jnp.float32),
            pltpu.VMEM((B,), jnp.int32),
            pltpu.VMEM((W, _ROWS_PER_W), jnp.int32),
            pltpu.VMEM((W, _ROWS_PER_W), jnp.int32),
            pltpu.VMEM((W, _ROWS_PER_W), jnp.float32),
        ],
    )(_sc_gather_body)
    return f(dist, order, gidx3)


def _negv_body(dist_ref, pid_row_ref, pid_all_ref, vn_ref, *, br):
    i0 = pl.program_id(0)
    d = dist_ref[...]
    mask = pid_row_ref[...] == pid_all_ref[...]
    j = lax.broadcasted_iota(jnp.int32, (br, B), 1)
    vn_ref[...] = _neg_side(d, mask, j, i0, br)


def _negv_path(dist, pid_col, pid_all):
    br = 64
    body = functools.partial(_negv_body, br=br)
    return pl.pallas_call(
        body,
        grid=(B // br,),
        in_specs=[
            pl.BlockSpec((br, B), lambda i: (i, 0)),
            pl.BlockSpec((br, 1), lambda i: (i, 0)),
            pl.BlockSpec((1, B), lambda i: (0, 0)),
        ],
        out_specs=pl.BlockSpec((br, 1), lambda i: (i, 0)),
        out_shape=jax.ShapeDtypeStruct((B, 1), jnp.float32),
    )(dist, pid_col, pid_all)


def _pos_body(col_ref, dval_ref, vn_ref, out_ref, *, br):
    i0 = pl.program_id(0)
    col = col_ref[...]
    dval = dval_ref[...]
    rw = lax.broadcasted_iota(jnp.int32, (br, W), 0) + i0 * np.int32(br)
    pp = lax.shift_left(rw, np.int32(12)) + col
    gp = _gumbel(_threefry_bits(pp, 123))
    score_p = gp + dval
    neg_inf = np.float32(-np.inf)
    m = jnp.max(score_p, axis=1, keepdims=True)
    selcol = jnp.min(
        jnp.where(score_p == m, col, np.int32(B)), axis=1, keepdims=True
    )
    vp = jnp.max(
        jnp.where(col == selcol, dval, neg_inf), axis=1, keepdims=True
    )
    out_ref[...] = _softplus(vp - vn_ref[...])


def _pos_path(colidx, dval, vn):
    br = 1024
    body = functools.partial(_pos_body, br=br)
    return pl.pallas_call(
        body,
        grid=(B // br,),
        in_specs=[
            pl.BlockSpec((br, W), lambda i: (i, 0)),
            pl.BlockSpec((br, W), lambda i: (i, 0)),
            pl.BlockSpec((br, 1), lambda i: (i, 0)),
        ],
        out_specs=pl.BlockSpec((br, 1), lambda i: (i, 0)),
        out_shape=jax.ShapeDtypeStruct((B, 1), jnp.float32),
    )(colidx, dval, vn)


@jax.jit
def kernel(dist, pids):
    pid_col = pids.reshape(B, 1)
    pid_all = pids.reshape(1, B)

    counts = jnp.zeros((64,), jnp.int32).at[pids].add(1)
    starts = jnp.cumsum(counts) - counts
    order = jnp.argsort(pids).astype(jnp.int32)
    start_i = starts[pids]
    cnt_i = counts[pids]
    k = jnp.arange(W, dtype=jnp.int32)
    gidx = (start_i[:, None] + jnp.minimum(k[None, :], cnt_i[:, None] - 1)
            ).astype(jnp.int32)
    gidx3 = gidx.reshape(_NW, _ROWS_PER_W, W).transpose(0, 2, 1)

    def fast(dist, pid_col, pid_all, gidx3, order):
        col3, dval3 = _sc_gather(dist, order, gidx3)
        vn = _negv_path(dist, pid_col, pid_all)
        colidx = col3.transpose(0, 2, 1).reshape(B, W)
        dval = dval3.transpose(0, 2, 1).reshape(B, W)
        return _pos_path(colidx, dval, vn)

    def slow(dist, pid_col, pid_all, gidx3, order):
        return _dense_path(dist, pid_col, pid_all)

    out = lax.cond(
        jnp.max(counts) <= W, fast, slow,
        dist, pid_col, pid_all, gidx3, order,
    )
    return out.reshape(B)


# speedup vs baseline: 1.0139x; 1.0139x over previous
import functools

import jax
import jax.numpy as jnp
import numpy as np
from jax import lax
from jax.experimental import pallas as pl
from jax.experimental.pallas import tpu as pltpu
from jax.experimental.pallas import tpu_sc as plsc

B = 4096
W = 128
_TINY = np.float32(np.finfo(np.float32).tiny)


def _rotl(x, d):
    return lax.shift_left(x, np.int32(d)) | lax.shift_right_logical(
        x, np.int32(32 - d)
    )


def _threefry_bits(p, seed):
    ks0 = np.int32(0)
    ks1 = np.int32(seed)
    ks2 = np.int32(0x1BD11BDA ^ seed)
    x0 = jnp.zeros_like(p)
    x1 = p + ks1

    def four(x0, x1, rots):
        for r in rots:
            x0 = x0 + x1
            x1 = _rotl(x1, r)
            x1 = x1 ^ x0
        return x0, x1

    r0 = (13, 15, 26, 6)
    r1 = (17, 29, 16, 24)
    x0, x1 = four(x0, x1, r0)
    x0, x1 = x0 + ks1, x1 + (ks2 + np.int32(1))
    x0, x1 = four(x0, x1, r1)
    x0, x1 = x0 + ks2, x1 + (ks0 + np.int32(2))
    x0, x1 = four(x0, x1, r0)
    x0, x1 = x0 + ks0, x1 + (ks1 + np.int32(3))
    x0, x1 = four(x0, x1, r1)
    x0, x1 = x0 + ks1, x1 + (ks2 + np.int32(4))
    x0, x1 = four(x0, x1, r0)
    x0, x1 = x0 + ks2, x1 + (ks0 + np.int32(5))
    return x0 ^ x1


def _gumbel(bits):
    fb = lax.shift_right_logical(bits, np.int32(9)) | np.int32(0x3F800000)
    floats = lax.bitcast_convert_type(fb, jnp.float32) - np.float32(1.0)
    u = jnp.maximum(_TINY, floats * (np.float32(1.0) - _TINY) + _TINY)
    return -jnp.log(-jnp.log(u))


def _softplus(x):
    return jnp.maximum(x, 0.0) + jnp.log1p(jnp.exp(-jnp.abs(x)))


def _neg_side(d, mask, j, i0, br):
    r = lax.broadcasted_iota(jnp.int32, (br, B), 0) + i0 * np.int32(br)
    p = lax.shift_left(r, np.int32(12)) + j
    gn = _gumbel(_threefry_bits(p, 456))
    neg_inf = np.float32(-np.inf)
    score = jnp.where(mask, neg_inf, gn + (-d))
    m = jnp.max(score, axis=1, keepdims=True)
    idx = jnp.min(jnp.where(score == m, j, np.int32(B)), axis=1, keepdims=True)
    return jnp.max(jnp.where(j == idx, d, neg_inf), axis=1, keepdims=True)


def _dense_body(dist_ref, pid_row_ref, pid_all_ref, out_ref, *, br):
    i0 = pl.program_id(0)
    d = dist_ref[...]
    mask = pid_row_ref[...] == pid_all_ref[...]
    j = lax.broadcasted_iota(jnp.int32, (br, B), 1)

    r = lax.broadcasted_iota(jnp.int32, (br, B), 0) + i0 * np.int32(br)
    p = lax.shift_left(r, np.int32(12)) + j
    neg_inf = np.float32(-np.inf)
    gp = _gumbel(_threefry_bits(p, 123))
    score_p = jnp.where(mask, gp + d, neg_inf)
    m = jnp.max(score_p, axis=1, keepdims=True)
    idx = jnp.min(
        jnp.where(score_p == m, j, np.int32(B)), axis=1, keepdims=True
    )
    vp = jnp.max(jnp.where(j == idx, d, neg_inf), axis=1, keepdims=True)

    vn = _neg_side(d, mask, j, i0, br)
    out_ref[...] = _softplus(vp - vn)


def _dense_path(dist, pid_col, pid_all):
    br = 256
    body = functools.partial(_dense_body, br=br)
    return pl.pallas_call(
        body,
        grid=(B // br,),
        in_specs=[
            pl.BlockSpec((br, B), lambda i: (i, 0)),
            pl.BlockSpec((br, 1), lambda i: (i, 0)),
            pl.BlockSpec((1, B), lambda i: (0, 0)),
        ],
        out_specs=pl.BlockSpec((br, 1), lambda i: (i, 0)),
        out_shape=jax.ShapeDtypeStruct((B, 1), jnp.float32),
    )(dist, pid_col, pid_all)


_NW = 32
_ROWS_PER_W = B // _NW
_CR = 16


def _sc_gather_body(dist_hbm, order_hbm, gidx_hbm, col_hbm, dval_hbm,
                    rowbuf, order_buf, gbuf, cbuf, dbuf):
    wid = lax.axis_index("s") * 2 + lax.axis_index("c")
    lane = lax.iota(jnp.int32, 16)
    pltpu.sync_copy(order_hbm, order_buf)
    pltpu.sync_copy(gidx_hbm.at[wid], gbuf)

    def chunk(c, _):
        r0 = wid * np.int32(_ROWS_PER_W) + c * np.int32(_CR)
        pltpu.sync_copy(dist_hbm.at[pl.ds(r0, _CR)], rowbuf)
        c16 = c * np.int32(_CR)

        def cand(k, _):
            g = gbuf[k, pl.ds(c16, _CR)]
            col = plsc.load_gather(order_buf, [g])
            dv = plsc.load_gather(rowbuf, [lane, col])
            cbuf[k, pl.ds(c16, _CR)] = col
            dbuf[k, pl.ds(c16, _CR)] = dv
            return 0

        lax.fori_loop(0, W, cand, 0, unroll=8)
        return 0

    lax.fori_loop(0, _ROWS_PER_W // _CR, chunk, 0)
    pltpu.sync_copy(cbuf, col_hbm.at[wid])
    pltpu.sync_copy(dbuf, dval_hbm.at[wid])


def _sc_gather(dist, order, gidx3):
    mesh = plsc.VectorSubcoreMesh(core_axis_name="c", subcore_axis_name="s")
    f = functools.partial(
        pl.kernel,
        mesh=mesh,
        compiler_params=pltpu.CompilerParams(needs_layout_passes=False),
        out_type=[
            jax.ShapeDtypeStruct((_NW, W, _ROWS_PER_W), jnp.int32),
            jax.ShapeDtypeStruct((_NW, W, _ROWS_PER_W), jnp.float32),
        ],
        scratch_types=[
            pltpu.VMEM((_CR, B), jnp.float32),
            pltpu.VMEM((B,), jnp.int32),
            pltpu.VMEM((W, _ROWS_PER_W), jnp.int32),
            pltpu.VMEM((W, _ROWS_PER_W), jnp.int32),
            pltpu.VMEM((W, _ROWS_PER_W), jnp.float32),
        ],
    )(_sc_gather_body)
    return f(dist, order, gidx3)


def _negv_body(dist_ref, pid_row_ref, pid_all_ref, vn_ref, *, br):
    i0 = pl.program_id(0)
    d = dist_ref[...]
    mask = pid_row_ref[...] == pid_all_ref[...]
    j = lax.broadcasted_iota(jnp.int32, (br, B), 1)
    vn_ref[...] = _neg_side(d, mask, j, i0, br)


def _negv_path(dist, pid_col, pid_all):
    br = 64
    body = functools.partial(_negv_body, br=br)
    return pl.pallas_call(
        body,
        grid=(B // br,),
        in_specs=[
            pl.BlockSpec((br, B), lambda i: (i, 0)),
            pl.BlockSpec((br, 1), lambda i: (i, 0)),
            pl.BlockSpec((1, B), lambda i: (0, 0)),
        ],
        out_specs=pl.BlockSpec((br, 1), lambda i: (i, 0)),
        out_shape=jax.ShapeDtypeStruct((B, 1), jnp.float32),
    )(dist, pid_col, pid_all)


def _pos_body(col_ref, dval_ref, vn_ref, out_ref, *, bw):
    i0 = pl.program_id(0)
    col = col_ref[...]
    dval = dval_ref[...]
    w = lax.broadcasted_iota(jnp.int32, (bw, W, _ROWS_PER_W), 0) \
        + i0 * np.int32(bw)
    r = lax.broadcasted_iota(jnp.int32, (bw, W, _ROWS_PER_W), 2)
    row = w * np.int32(_ROWS_PER_W) + r
    pp = lax.shift_left(row, np.int32(12)) + col
    gp = _gumbel(_threefry_bits(pp, 123))
    score_p = gp + dval
    neg_inf = np.float32(-np.inf)
    m = jnp.max(score_p, axis=1, keepdims=True)
    selcol = jnp.min(
        jnp.where(score_p == m, col, np.int32(B)), axis=1, keepdims=True
    )
    vp = jnp.max(
        jnp.where(col == selcol, dval, neg_inf), axis=1, keepdims=True
    )
    out_ref[...] = _softplus(vp - vn_ref[...])


def _pos_path(col3, dval3, vn):
    bw = 4
    body = functools.partial(_pos_body, bw=bw)
    out = pl.pallas_call(
        body,
        grid=(_NW // bw,),
        in_specs=[
            pl.BlockSpec((bw, W, _ROWS_PER_W), lambda i: (i, 0, 0)),
            pl.BlockSpec((bw, W, _ROWS_PER_W), lambda i: (i, 0, 0)),
            pl.BlockSpec((bw, 1, _ROWS_PER_W), lambda i: (i, 0, 0)),
        ],
        out_specs=pl.BlockSpec((bw, 1, _ROWS_PER_W), lambda i: (i, 0, 0)),
        out_shape=jax.ShapeDtypeStruct((_NW, 1, _ROWS_PER_W), jnp.float32),
    )(col3, dval3, vn.reshape(_NW, 1, _ROWS_PER_W))
    return out.reshape(B)


@jax.jit
def kernel(dist, pids):
    pid_col = pids.reshape(B, 1)
    pid_all = pids.reshape(1, B)

    counts = jnp.zeros((64,), jnp.int32).at[pids].add(1)
    starts = jnp.cumsum(counts) - counts
    order = jnp.argsort(pids).astype(jnp.int32)
    start_i = starts[pids]
    cnt_i = counts[pids]
    k = jnp.arange(W, dtype=jnp.int32)
    gidx = (start_i[:, None] + jnp.minimum(k[None, :], cnt_i[:, None] - 1)
            ).astype(jnp.int32)
    gidx3 = gidx.reshape(_NW, _ROWS_PER_W, W).transpose(0, 2, 1)

    def fast(dist, pid_col, pid_all, gidx3, order):
        col3, dval3 = _sc_gather(dist, order, gidx3)
        vn = _negv_path(dist, pid_col, pid_all)
        return _pos_path(col3, dval3, vn)

    def slow(dist, pid_col, pid_all, gidx3, order):
        return _dense_path(dist, pid_col, pid_all).reshape(B)

    return lax.cond(
        jnp.max(counts) <= W, fast, slow,
        dist, pid_col, pid_all, gidx3, order,
    )


# speedup vs baseline: 1.0173x; 1.0034x over previous
import functools

import jax
import jax.numpy as jnp
import numpy as np
from jax import lax
from jax.experimental import pallas as pl
from jax.experimental.pallas import tpu as pltpu
from jax.experimental.pallas import tpu_sc as plsc

B = 4096
W = 128
_TINY = np.float32(np.finfo(np.float32).tiny)


def _rotl(x, d):
    return lax.shift_left(x, np.int32(d)) | lax.shift_right_logical(
        x, np.int32(32 - d)
    )


def _threefry_bits(x1, seed):
    ks0 = np.int32(0)
    ks1 = np.int32(seed)
    ks2 = np.int32(0x1BD11BDA ^ seed)

    def four(x0, x1, rots):
        for r in rots:
            x0 = x0 + x1
            x1 = _rotl(x1, r)
            x1 = x1 ^ x0
        return x0, x1

    r0 = (13, 15, 26, 6)
    r1 = (17, 29, 16, 24)
    x0 = x1
    x1 = _rotl(x1, 13) ^ x0
    x0, x1 = four(x0, x1, r0[1:])
    x0, x1 = x0 + ks1, x1 + (ks2 + np.int32(1))
    x0, x1 = four(x0, x1, r1)
    x0, x1 = x0 + ks2, x1 + (ks0 + np.int32(2))
    x0, x1 = four(x0, x1, r0)
    x0, x1 = x0 + ks0, x1 + (ks1 + np.int32(3))
    x0, x1 = four(x0, x1, r1)
    x0, x1 = x0 + ks1, x1 + (ks2 + np.int32(4))
    x0, x1 = four(x0, x1, r0)
    x0, x1 = x0 + ks2, x1 + (ks0 + np.int32(5))
    return x0 ^ x1


def _gumbel(bits):
    fb = lax.shift_right_logical(bits, np.int32(9)) | np.int32(0x3F800000)
    floats = lax.bitcast_convert_type(fb, jnp.float32) - np.float32(1.0)
    u = jnp.maximum(_TINY, floats * (np.float32(1.0) - _TINY) + _TINY)
    return -jnp.log(-jnp.log(u))


def _softplus(x):
    return jnp.maximum(x, 0.0) + jnp.log1p(jnp.exp(-jnp.abs(x)))


def _neg_side(d, mask, j, i0, br):
    r = lax.broadcasted_iota(jnp.int32, (br, B), 0) + i0 * np.int32(br)
    x1n = lax.shift_left(r, np.int32(12)) + (j + np.int32(456))
    gn = _gumbel(_threefry_bits(x1n, 456))
    neg_inf = np.float32(-np.inf)
    score = jnp.where(mask, neg_inf, gn + (-d))
    m = jnp.max(score, axis=1, keepdims=True)
    idx = jnp.min(jnp.where(score == m, j, np.int32(B)), axis=1, keepdims=True)
    return jnp.max(jnp.where(j == idx, d, neg_inf), axis=1, keepdims=True)


def _dense_body(dist_ref, pid_row_ref, pid_all_ref, out_ref, *, br):
    i0 = pl.program_id(0)
    d = dist_ref[...]
    mask = pid_row_ref[...] == pid_all_ref[...]
    j = lax.broadcasted_iota(jnp.int32, (br, B), 1)

    r = lax.broadcasted_iota(jnp.int32, (br, B), 0) + i0 * np.int32(br)
    x1p = lax.shift_left(r, np.int32(12)) + (j + np.int32(123))
    neg_inf = np.float32(-np.inf)
    gp = _gumbel(_threefry_bits(x1p, 123))
    score_p = jnp.where(mask, gp + d, neg_inf)
    m = jnp.max(score_p, axis=1, keepdims=True)
    idx = jnp.min(
        jnp.where(score_p == m, j, np.int32(B)), axis=1, keepdims=True
    )
    vp = jnp.max(jnp.where(j == idx, d, neg_inf), axis=1, keepdims=True)

    vn = _neg_side(d, mask, j, i0, br)
    out_ref[...] = _softplus(vp - vn)


def _dense_path(dist, pid_col, pid_all):
    br = 256
    body = functools.partial(_dense_body, br=br)
    return pl.pallas_call(
        body,
        grid=(B // br,),
        in_specs=[
            pl.BlockSpec((br, B), lambda i: (i, 0)),
            pl.BlockSpec((br, 1), lambda i: (i, 0)),
            pl.BlockSpec((1, B), lambda i: (0, 0)),
        ],
        out_specs=pl.BlockSpec((br, 1), lambda i: (i, 0)),
        out_shape=jax.ShapeDtypeStruct((B, 1), jnp.float32),
    )(dist, pid_col, pid_all)


_NW = 32
_ROWS_PER_W = B // _NW
_CR = 16


def _sc_gather_body(dist_hbm, order_hbm, gidx_hbm, col_hbm, dval_hbm,
                    rowbuf, order_buf, gbuf, cbuf, dbuf):
    wid = lax.axis_index("s") * 2 + lax.axis_index("c")
    lane = lax.iota(jnp.int32, 16)
    pltpu.sync_copy(order_hbm, order_buf)
    pltpu.sync_copy(gidx_hbm.at[wid], gbuf)

    def chunk(c, _):
        r0 = wid * np.int32(_ROWS_PER_W) + c * np.int32(_CR)
        pltpu.sync_copy(dist_hbm.at[pl.ds(r0, _CR)], rowbuf)
        c16 = c * np.int32(_CR)

        def cand(k, _):
            g = gbuf[k, pl.ds(c16, _CR)]
            col = plsc.load_gather(order_buf, [g])
            dv = plsc.load_gather(rowbuf, [lane, col])
            cbuf[k, pl.ds(c16, _CR)] = col
            dbuf[k, pl.ds(c16, _CR)] = dv
            return 0

        lax.fori_loop(0, W, cand, 0, unroll=8)
        return 0

    lax.fori_loop(0, _ROWS_PER_W // _CR, chunk, 0)
    pltpu.sync_copy(cbuf, col_hbm.at[wid])
    pltpu.sync_copy(dbuf, dval_hbm.at[wid])


def _sc_gather(dist, order, gidx3):
    mesh = plsc.VectorSubcoreMesh(core_axis_name="c", subcore_axis_name="s")
    f = functools.partial(
        pl.kernel,
        mesh=mesh,
        compiler_params=pltpu.CompilerParams(needs_layout_passes=False),
        out_type=[
            jax.ShapeDtypeStruct((_NW, W, _ROWS_PER_W), jnp.int32),
            jax.ShapeDtypeStruct((_NW, W, _ROWS_PER_W), jnp.float32),
        ],
        scratch_types=[
            pltpu.VMEM((_CR, B), jnp.float32),
            pltpu.VMEM((B,), jnp.int32),
            pltpu.VMEM((W, _ROWS_PER_W), jnp.int32),
            pltpu.VMEM((W, _ROWS_PER_W), jnp.int32),
            pltpu.VMEM((W, _ROWS_PER_W), jnp.float32),
        ],
    )(_sc_gather_body)
    return f(dist, order, gidx3)


def _negv_body(dist_ref, pid_row_ref, pid_all_ref, vn_ref, *, br):
    i0 = pl.program_id(0)
    d = dist_ref[...]
    mask = pid_row_ref[...] == pid_all_ref[...]
    j = lax.broadcasted_iota(jnp.int32, (br, B), 1)
    vn_ref[...] = _neg_side(d, mask, j, i0, br)


def _negv_path(dist, pid_col, pid_all):
    br = 64
    body = functools.partial(_negv_body, br=br)
    return pl.pallas_call(
        body,
        grid=(B // br,),
        in_specs=[
            pl.BlockSpec((br, B), lambda i: (i, 0)),
            pl.BlockSpec((br, 1), lambda i: (i, 0)),
            pl.BlockSpec((1, B), lambda i: (0, 0)),
        ],
        out_specs=pl.BlockSpec((br, 1), lambda i: (i, 0)),
        out_shape=jax.ShapeDtypeStruct((B, 1), jnp.float32),
    )(dist, pid_col, pid_all)


def _pos_body(col_ref, dval_ref, vn_ref, out_ref, *, bw):
    i0 = pl.program_id(0)
    col = col_ref[...]
    dval = dval_ref[...]
    w = lax.broadcasted_iota(jnp.int32, (bw, W, _ROWS_PER_W), 0) \
        + i0 * np.int32(bw)
    r = lax.broadcasted_iota(jnp.int32, (bw, W, _ROWS_PER_W), 2)
    row = w * np.int32(_ROWS_PER_W) + r
    pp = lax.shift_left(row, np.int32(12)) + (col + np.int32(123))
    gp = _gumbel(_threefry_bits(pp, 123))
    score_p = gp + dval
    neg_inf = np.float32(-np.inf)
    m = jnp.max(score_p, axis=1, keepdims=True)
    selcol = jnp.min(
        jnp.where(score_p == m, col, np.int32(B)), axis=1, keepdims=True
    )
    vp = jnp.max(
        jnp.where(col == selcol, dval, neg_inf), axis=1, keepdims=True
    )
    out_ref[...] = _softplus(vp - vn_ref[...])


def _pos_path(col3, dval3, vn):
    bw = 4
    body = functools.partial(_pos_body, bw=bw)
    out = pl.pallas_call(
        body,
        grid=(_NW // bw,),
        in_specs=[
            pl.BlockSpec((bw, W, _ROWS_PER_W), lambda i: (i, 0, 0)),
            pl.BlockSpec((bw, W, _ROWS_PER_W), lambda i: (i, 0, 0)),
            pl.BlockSpec((bw, 1, _ROWS_PER_W), lambda i: (i, 0, 0)),
        ],
        out_specs=pl.BlockSpec((bw, 1, _ROWS_PER_W), lambda i: (i, 0, 0)),
        out_shape=jax.ShapeDtypeStruct((_NW, 1, _ROWS_PER_W), jnp.float32),
    )(col3, dval3, vn.reshape(_NW, 1, _ROWS_PER_W))
    return out.reshape(B)


@jax.jit
def kernel(dist, pids):
    pid_col = pids.reshape(B, 1)
    pid_all = pids.reshape(1, B)

    counts = jnp.zeros((64,), jnp.int32).at[pids].add(1)
    starts = jnp.cumsum(counts) - counts
    order = jnp.argsort(pids).astype(jnp.int32)
    start_i = starts[pids]
    cnt_i = counts[pids]
    k = jnp.arange(W, dtype=jnp.int32)
    gidx = (start_i[:, None] + jnp.minimum(k[None, :], cnt_i[:, None] - 1)
            ).astype(jnp.int32)
    gidx3 = gidx.reshape(_NW, _ROWS_PER_W, W).transpose(0, 2, 1)

    def fast(dist, pid_col, pid_all, gidx3, order):
        col3, dval3 = _sc_gather(dist, order, gidx3)
        vn = _negv_path(dist, pid_col, pid_all)
        return _pos_path(col3, dval3, vn)

    def slow(dist, pid_col, pid_all, gidx3, order):
        return _dense_path(dist, pid_col, pid_all).reshape(B)

    return lax.cond(
        jnp.max(counts) <= W, fast, slow,
        dist, pid_col, pid_all, gidx3, order,
    )


# speedup vs baseline: 1.0304x; 1.0129x over previous
import functools

import jax
import jax.numpy as jnp
import numpy as np
from jax import lax
from jax.experimental import pallas as pl
from jax.experimental.pallas import tpu as pltpu
from jax.experimental.pallas import tpu_sc as plsc

B = 4096
W = 128
_TINY = np.float32(np.finfo(np.float32).tiny)


def _rotl(x, d):
    return lax.shift_left(x, np.int32(d)) | lax.shift_right_logical(
        x, np.int32(32 - d)
    )


def _threefry_bits(x1, seed):
    ks0 = np.int32(0)
    ks1 = np.int32(seed)
    ks2 = np.int32(0x1BD11BDA ^ seed)

    def four(x0, x1, rots):
        for r in rots:
            x0 = x0 + x1
            x1 = _rotl(x1, r)
            x1 = x1 ^ x0
        return x0, x1

    r0 = (13, 15, 26, 6)
    r1 = (17, 29, 16, 24)
    x0 = x1
    x1 = _rotl(x1, 13) ^ x0
    x0, x1 = four(x0, x1, r0[1:])
    x0, x1 = x0 + ks1, x1 + (ks2 + np.int32(1))
    x0, x1 = four(x0, x1, r1)
    x0, x1 = x0 + ks2, x1 + (ks0 + np.int32(2))
    x0, x1 = four(x0, x1, r0)
    x0, x1 = x0 + ks0, x1 + (ks1 + np.int32(3))
    x0, x1 = four(x0, x1, r1)
    x0, x1 = x0 + ks1, x1 + (ks2 + np.int32(4))
    x0, x1 = four(x0, x1, r0)
    x0, x1 = x0 + ks2, x1 + (ks0 + np.int32(5))
    return x0 ^ x1


def _gumbel(bits):
    fb = lax.shift_right_logical(bits, np.int32(9)) | np.int32(0x3F800000)
    floats = lax.bitcast_convert_type(fb, jnp.float32) - np.float32(1.0)
    u = jnp.maximum(_TINY, floats * (np.float32(1.0) - _TINY) + _TINY)
    return -jnp.log(-jnp.log(u))


def _softplus(x):
    return jnp.maximum(x, 0.0) + jnp.log1p(jnp.exp(-jnp.abs(x)))


def _neg_side(d, mask, j, i0, br):
    r = lax.broadcasted_iota(jnp.int32, (br, B), 0) + i0 * np.int32(br)
    x1n = lax.shift_left(r, np.int32(12)) + (j + np.int32(456))
    gn = _gumbel(_threefry_bits(x1n, 456))
    neg_inf = np.float32(-np.inf)
    score = jnp.where(mask, neg_inf, gn + (-d))
    m = jnp.max(score, axis=1, keepdims=True)
    idx = jnp.min(jnp.where(score == m, j, np.int32(B)), axis=1, keepdims=True)
    return jnp.max(jnp.where(j == idx, d, neg_inf), axis=1, keepdims=True)


def _dense_body(dist_ref, pid_row_ref, pid_all_ref, out_ref, *, br):
    i0 = pl.program_id(0)
    d = dist_ref[...]
    mask = pid_row_ref[...] == pid_all_ref[...]
    j = lax.broadcasted_iota(jnp.int32, (br, B), 1)

    r = lax.broadcasted_iota(jnp.int32, (br, B), 0) + i0 * np.int32(br)
    x1p = lax.shift_left(r, np.int32(12)) + (j + np.int32(123))
    neg_inf = np.float32(-np.inf)
    gp = _gumbel(_threefry_bits(x1p, 123))
    score_p = jnp.where(mask, gp + d, neg_inf)
    m = jnp.max(score_p, axis=1, keepdims=True)
    idx = jnp.min(
        jnp.where(score_p == m, j, np.int32(B)), axis=1, keepdims=True
    )
    vp = jnp.max(jnp.where(j == idx, d, neg_inf), axis=1, keepdims=True)

    vn = _neg_side(d, mask, j, i0, br)
    out_ref[...] = _softplus(vp - vn)


def _dense_path(dist, pid_col, pid_all):
    br = 256
    body = functools.partial(_dense_body, br=br)
    return pl.pallas_call(
        body,
        grid=(B // br,),
        in_specs=[
            pl.BlockSpec((br, B), lambda i: (i, 0)),
            pl.BlockSpec((br, 1), lambda i: (i, 0)),
            pl.BlockSpec((1, B), lambda i: (0, 0)),
        ],
        out_specs=pl.BlockSpec((br, 1), lambda i: (i, 0)),
        out_shape=jax.ShapeDtypeStruct((B, 1), jnp.float32),
    )(dist, pid_col, pid_all)


_NW = 32
_ROWS_PER_W = B // _NW
_CR = 16


def _sc_gather_body(dist_hbm, order_hbm, gidx_hbm, col_hbm, dval_hbm,
                    rowbuf, order_buf, gbuf, cbuf, dbuf):
    wid = lax.axis_index("s") * 2 + lax.axis_index("c")
    lane = lax.iota(jnp.int32, 16)
    pltpu.sync_copy(order_hbm, order_buf)
    pltpu.sync_copy(gidx_hbm.at[wid], gbuf)

    def chunk(c, _):
        r0 = wid * np.int32(_ROWS_PER_W) + c * np.int32(_CR)
        pltpu.sync_copy(dist_hbm.at[pl.ds(r0, _CR)], rowbuf)
        c16 = c * np.int32(_CR)

        def cand(k, _):
            g = gbuf[k, pl.ds(c16, _CR)]
            col = plsc.load_gather(order_buf, [g])
            dv = plsc.load_gather(rowbuf, [lane, col])
            cbuf[k, pl.ds(c16, _CR)] = col
            dbuf[k, pl.ds(c16, _CR)] = dv
            return 0

        lax.fori_loop(0, W, cand, 0, unroll=8)
        return 0

    lax.fori_loop(0, _ROWS_PER_W // _CR, chunk, 0)
    pltpu.sync_copy(cbuf, col_hbm.at[wid])
    pltpu.sync_copy(dbuf, dval_hbm.at[wid])


def _sc_gather(dist, order, gidx3):
    mesh = plsc.VectorSubcoreMesh(core_axis_name="c", subcore_axis_name="s")
    f = functools.partial(
        pl.kernel,
        mesh=mesh,
        compiler_params=pltpu.CompilerParams(needs_layout_passes=False),
        out_type=[
            jax.ShapeDtypeStruct((_NW, W, _ROWS_PER_W), jnp.int32),
            jax.ShapeDtypeStruct((_NW, W, _ROWS_PER_W), jnp.float32),
        ],
        scratch_types=[
            pltpu.VMEM((_CR, B), jnp.float32),
            pltpu.VMEM((B,), jnp.int32),
            pltpu.VMEM((W, _ROWS_PER_W), jnp.int32),
            pltpu.VMEM((W, _ROWS_PER_W), jnp.int32),
            pltpu.VMEM((W, _ROWS_PER_W), jnp.float32),
        ],
    )(_sc_gather_body)
    return f(dist, order, gidx3)


def _negv_body(dist_ref, pid_row_ref, pid_all_ref, vn_ref, *, br):
    i0 = pl.program_id(0)
    d = dist_ref[...]
    mask = pid_row_ref[...] == pid_all_ref[...]
    j = lax.broadcasted_iota(jnp.int32, (br, B), 1)
    vn_ref[...] = _neg_side(d, mask, j, i0, br)


def _negv_path(dist, pid_col, pid_all):
    br = 128
    body = functools.partial(_negv_body, br=br)
    return pl.pallas_call(
        body,
        grid=(B // br,),
        in_specs=[
            pl.BlockSpec((br, B), lambda i: (i, 0)),
            pl.BlockSpec((br, 1), lambda i: (i, 0)),
            pl.BlockSpec((1, B), lambda i: (0, 0)),
        ],
        out_specs=pl.BlockSpec((br, 1), lambda i: (i, 0)),
        out_shape=jax.ShapeDtypeStruct((B, 1), jnp.float32),
    )(dist, pid_col, pid_all)


def _pos_body(col_ref, dval_ref, vn_ref, out_ref, *, bw):
    i0 = pl.program_id(0)
    col = col_ref[...]
    dval = dval_ref[...]
    w = lax.broadcasted_iota(jnp.int32, (bw, W, _ROWS_PER_W), 0) \
        + i0 * np.int32(bw)
    r = lax.broadcasted_iota(jnp.int32, (bw, W, _ROWS_PER_W), 2)
    row = w * np.int32(_ROWS_PER_W) + r
    pp = lax.shift_left(row, np.int32(12)) + (col + np.int32(123))
    gp = _gumbel(_threefry_bits(pp, 123))
    score_p = gp + dval
    neg_inf = np.float32(-np.inf)
    m = jnp.max(score_p, axis=1, keepdims=True)
    selcol = jnp.min(
        jnp.where(score_p == m, col, np.int32(B)), axis=1, keepdims=True
    )
    vp = jnp.max(
        jnp.where(col == selcol, dval, neg_inf), axis=1, keepdims=True
    )
    out_ref[...] = _softplus(vp - vn_ref[...])


def _pos_path(col3, dval3, vn):
    bw = 4
    body = functools.partial(_pos_body, bw=bw)
    out = pl.pallas_call(
        body,
        grid=(_NW // bw,),
        in_specs=[
            pl.BlockSpec((bw, W, _ROWS_PER_W), lambda i: (i, 0, 0)),
            pl.BlockSpec((bw, W, _ROWS_PER_W), lambda i: (i, 0, 0)),
            pl.BlockSpec((bw, 1, _ROWS_PER_W), lambda i: (i, 0, 0)),
        ],
        out_specs=pl.BlockSpec((bw, 1, _ROWS_PER_W), lambda i: (i, 0, 0)),
        out_shape=jax.ShapeDtypeStruct((_NW, 1, _ROWS_PER_W), jnp.float32),
    )(col3, dval3, vn.reshape(_NW, 1, _ROWS_PER_W))
    return out.reshape(B)


@jax.jit
def kernel(dist, pids):
    pid_col = pids.reshape(B, 1)
    pid_all = pids.reshape(1, B)

    counts = jnp.zeros((64,), jnp.int32).at[pids].add(1)
    starts = jnp.cumsum(counts) - counts
    order = jnp.argsort(pids).astype(jnp.int32)
    start_i = starts[pids]
    cnt_i = counts[pids]
    k = jnp.arange(W, dtype=jnp.int32)
    gidx = (start_i[:, None] + jnp.minimum(k[None, :], cnt_i[:, None] - 1)
            ).astype(jnp.int32)
    gidx3 = gidx.reshape(_NW, _ROWS_PER_W, W).transpose(0, 2, 1)

    def fast(dist, pid_col, pid_all, gidx3, order):
        col3, dval3 = _sc_gather(dist, order, gidx3)
        vn = _negv_path(dist, pid_col, pid_all)
        return _pos_path(col3, dval3, vn)

    def slow(dist, pid_col, pid_all, gidx3, order):
        return _dense_path(dist, pid_col, pid_all).reshape(B)

    return lax.cond(
        jnp.max(counts) <= W, fast, slow,
        dist, pid_col, pid_all, gidx3, order,
    )
